# baseline (speedup 1.0000x reference)
import jax
import jax.numpy as jnp
from jax.experimental import pallas as pl
from jax.experimental.pallas import tpu as pltpu

_K = 8
_N = 16384
_D = 2048
_N_CHUNKS = 8
_FIX_ROWS = 1024


def _body(x_ref, y_hbm, out_hbm, scratch, sems, fix_sem):
    rows_per_chunk = _N // _N_CHUNKS

    for c in range(_N_CHUNKS):
        sl = pl.ds(c * rows_per_chunk, rows_per_chunk)
        pltpu.make_async_copy(y_hbm.at[sl, :], out_hbm.at[sl, :], sems.at[c]).start()

    labels = x_ref[:, :]
    bad = (labels < 0) | (labels >= _K)

    for c in range(_N_CHUNKS):
        sl = pl.ds(c * rows_per_chunk, rows_per_chunk)
        pltpu.make_async_copy(y_hbm.at[sl, :], out_hbm.at[sl, :], sems.at[c]).wait()

    for b in range(_N // _FIX_ROWS):
        blk_bad = bad[b * _FIX_ROWS:(b + 1) * _FIX_ROWS, :]
        n_bad = jnp.sum(blk_bad.astype(jnp.int32))

        @pl.when(n_bad > 0)
        def _fix(b=b, blk_bad=blk_bad):
            sl = pl.ds(b * _FIX_ROWS, _FIX_ROWS)
            cp_in = pltpu.make_async_copy(out_hbm.at[sl, :], scratch, fix_sem)
            cp_in.start()
            cp_in.wait()
            scratch[:, :] = jnp.where(blk_bad, 0.0, scratch[:, :])
            cp_out = pltpu.make_async_copy(scratch, out_hbm.at[sl, :], fix_sem)
            cp_out.start()
            cp_out.wait()


def kernel(x, y):
    n, d = y.shape
    x2 = x.astype(jnp.int32).reshape(n, 1)
    return pl.pallas_call(
        _body,
        in_specs=[
            pl.BlockSpec(memory_space=pltpu.MemorySpace.VMEM),
            pl.BlockSpec(memory_space=pl.ANY),
        ],
        out_specs=pl.BlockSpec(memory_space=pl.ANY),
        out_shape=jax.ShapeDtypeStruct((n, d), y.dtype),
        scratch_shapes=[
            pltpu.VMEM((_FIX_ROWS, _D), jnp.float32),
            pltpu.SemaphoreType.DMA((_N_CHUNKS,)),
            pltpu.SemaphoreType.DMA,
        ],
    )(x2, y)


# speedup vs baseline: 44.0118x; 44.0118x over previous
import jax
import jax.numpy as jnp
from jax.experimental import pallas as pl
from jax.experimental.pallas import tpu as pltpu

_K = 8
_ROWS_PER_BLOCK = 512


def _select_block(x_ref, y_ref, out_ref):
    labels = x_ref[:, :]
    mask = (labels >= 0) & (labels < _K)
    out_ref[:, :] = jnp.where(mask, y_ref[:, :], 0.0)


def kernel(x, y):
    n, d = y.shape
    r = _ROWS_PER_BLOCK
    grid = n // r
    x2 = x.astype(jnp.int32).reshape(n, 1)
    return pl.pallas_call(
        _select_block,
        grid=(grid,),
        in_specs=[
            pl.BlockSpec((r, 1), lambda i: (i, 0)),
            pl.BlockSpec((r, d), lambda i: (i, 0)),
        ],
        out_specs=pl.BlockSpec((r, d), lambda i: (i, 0)),
        out_shape=jax.ShapeDtypeStruct((n, d), y.dtype),
        compiler_params=pltpu.CompilerParams(
            dimension_semantics=("parallel",),
        ),
    )(x2, y)


# speedup vs baseline: 49.2639x; 1.1193x over previous
import jax
import jax.numpy as jnp
from jax.experimental import pallas as pl
from jax.experimental.pallas import tpu as pltpu

_K = 8
_R = 1024


def _body(x_vmem, x_smem, y_ref, out_ref):
    b = pl.program_id(0)
    labels = x_vmem[:, :]
    n_bad = jnp.sum(((labels < 0) | (labels >= _K)).astype(jnp.int32))

    out_ref[:, :] = y_ref[:, :]

    @pl.when(n_bad > 0)
    def _fixup():
        def zero_bad_row(i, _):
            lab = x_smem[b * _R + i]

            @pl.when((lab < 0) | (lab >= _K))
            def _z():
                out_ref[pl.ds(i, 1), :] = jnp.zeros((1, out_ref.shape[1]),
                                                    out_ref.dtype)
            return _
        jax.lax.fori_loop(0, _R, zero_bad_row, 0)


def kernel(x, y):
    n, d = y.shape
    grid = n // _R
    xi = x.astype(jnp.int32)
    x2 = xi.reshape(n // 128, 128)
    return pl.pallas_call(
        _body,
        grid=(grid,),
        in_specs=[
            pl.BlockSpec((_R // 128, 128), lambda i: (i, 0)),
            pl.BlockSpec(memory_space=pltpu.MemorySpace.SMEM),
            pl.BlockSpec((_R, d), lambda i: (i, 0)),
        ],
        out_specs=pl.BlockSpec((_R, d), lambda i: (i, 0)),
        out_shape=jax.ShapeDtypeStruct((n, d), y.dtype),
        compiler_params=pltpu.CompilerParams(
            dimension_semantics=("arbitrary",),
        ),
    )(x2, xi, y)
